# baseline (speedup 1.0000x reference)
import functools

import jax
import jax.numpy as jnp
from jax import lax
from jax.experimental import pallas as pl
from jax.experimental.pallas import tpu as pltpu
from jax.experimental.pallas import tpu_sc as plsc

_NB = 16
_C = 16


def _sc_gather(table, idx):
    n = idx.shape[0]
    nw = 32
    per_w = n // nw
    ch = 5000
    nch = per_w // ch
    mesh = plsc.VectorSubcoreMesh(core_axis_name="c", subcore_axis_name="s")

    @functools.partial(
        pl.kernel,
        mesh=mesh,
        compiler_params=pltpu.CompilerParams(use_tc_tiling_on_sc=False),
        out_type=jax.ShapeDtypeStruct((n, _C), jnp.float32),
        scratch_types=[
            pltpu.VMEM((ch,), jnp.int32),
            pltpu.VMEM((ch, _C), jnp.float32),
            pltpu.SemaphoreType.DMA,
        ],
    )
    def k(table_hbm, idx_hbm, out_hbm, idx_v, rows_v, sem):
        wid = lax.axis_index("s") * 2 + lax.axis_index("c")
        for t in range(nch):
            base = wid * per_w + t * ch
            pltpu.sync_copy(idx_hbm.at[pl.ds(base, ch)], idx_v)
            pltpu.async_copy(table_hbm.at[idx_v], rows_v, sem).wait()
            pltpu.sync_copy(rows_v, out_hbm.at[pl.ds(base, ch)])

    return k(table, idx)


def _tc_body(px_ref, x_ref, adj_ref, wf_ref, tm_ref, gt_ref, gs_ref, o_ref):
    px = px_ref[0]
    xb = x_ref[0]
    adjb = adj_ref[...]
    wf = wf_ref[...]
    tm = tm_ref[...]
    gt = gt_ref[...]
    gs = gs_ref[...]
    f32 = jnp.float32

    m = (adjb != 0).astype(f32)
    xt = jnp.dot(xb, tm, preferred_element_type=f32)
    e = jnp.exp(xt - px)
    s = jnp.dot(e, gs, preferred_element_type=f32)
    rt = jnp.dot(1.0 / s, gt, preferred_element_type=f32)
    q = e * rt

    acc = None
    for nn in range(_NB):
        sl = slice(nn * _C, (nn + 1) * _C)
        term = (jnp.dot(q[:, sl], gt, preferred_element_type=f32) *
                jnp.dot(px[:, sl], tm, preferred_element_type=f32))
        acc = term if acc is None else acc + term

    out = jnp.dot(acc, wf, preferred_element_type=f32)
    cnt = jnp.sum(m, axis=1, keepdims=True)
    inv = jnp.where(cnt > 0.0, 1.0 / cnt, 0.0)
    o_ref[0] = jnp.maximum(out * inv, 0.0)


def _tc_dense(pxf, x, adj, wf, tm, gt, gs, interpret=False):
    b, v, _ = pxf.shape
    vb = 1000
    nj = v // vb
    grid = (b, nj)
    return pl.pallas_call(
        _tc_body,
        grid=grid,
        in_specs=[
            pl.BlockSpec((1, vb, _NB * _C), lambda i, j: (i, j, 0)),
            pl.BlockSpec((1, vb, _C), lambda i, j: (i, j, 0)),
            pl.BlockSpec((vb, _NB), lambda i, j: (j, 0)),
            pl.BlockSpec((_NB * _C, _C), lambda i, j: (0, 0)),
            pl.BlockSpec((_C, _NB * _C), lambda i, j: (0, 0)),
            pl.BlockSpec((_C, _NB * _C), lambda i, j: (0, 0)),
            pl.BlockSpec((_NB * _C, _C), lambda i, j: (0, 0)),
        ],
        out_specs=pl.BlockSpec((1, vb, _C), lambda i, j: (i, j, 0)),
        out_shape=jax.ShapeDtypeStruct((b, v, _C), jnp.float32),
        interpret=interpret,
    )(pxf, x, adj, wf, tm, gt, gs)


def kernel(x, adj, W, u):
    del u
    b, v, c = x.shape
    nb = adj.shape[1]
    out_f = W.shape[2]

    idx = adj.reshape(-1)
    wf = W.transpose(1, 0, 2).reshape(nb * c, out_f)
    eye = jnp.eye(c, dtype=jnp.float32)
    tm = jnp.tile(eye, (1, nb))
    gt = jnp.repeat(eye, nb, axis=1)
    gs = gt.T

    pad = jnp.zeros((1, c), x.dtype)
    outs = []
    for bi in range(b):
        table = jnp.concatenate([pad, x[bi]], axis=0)
        px = _sc_gather(table, idx)
        pxf = px.reshape(1, v, nb * c)
        outs.append(_tc_dense(pxf, x[bi:bi + 1], adj, wf, tm, gt, gs))
    return jnp.concatenate(outs, axis=0)


# speedup vs baseline: 1.0462x; 1.0462x over previous
import functools

import jax
import jax.numpy as jnp
from jax import lax
from jax.experimental import pallas as pl
from jax.experimental.pallas import tpu as pltpu
from jax.experimental.pallas import tpu_sc as plsc

_NB = 16
_C = 16


def _sc_gather(table, idx):
    n = idx.shape[0]
    nw = 32
    per_w = n // nw
    ch = 5000
    nch = per_w // ch
    mesh = plsc.VectorSubcoreMesh(core_axis_name="c", subcore_axis_name="s")

    @functools.partial(
        pl.kernel,
        mesh=mesh,
        compiler_params=pltpu.CompilerParams(use_tc_tiling_on_sc=False),
        out_type=jax.ShapeDtypeStruct((n, _C), jnp.float32),
        scratch_types=[
            pltpu.VMEM((ch,), jnp.int32),
            pltpu.VMEM((ch, _C), jnp.float32),
            pltpu.SemaphoreType.DMA,
        ],
    )
    def k(table_hbm, idx_hbm, out_hbm, idx_v, rows_v, sem):
        wid = lax.axis_index("s") * 2 + lax.axis_index("c")
        for t in range(nch):
            base = wid * per_w + t * ch
            pltpu.sync_copy(idx_hbm.at[pl.ds(base, ch)], idx_v)
            pltpu.async_copy(table_hbm.at[idx_v], rows_v, sem).wait()
            pltpu.sync_copy(rows_v, out_hbm.at[pl.ds(base, ch)])

    return k(table, idx)


def _tc_body(pxa_ref, pxb_ref, x_ref, adj_ref, wf_ref, tm_ref, gt_ref,
             t8_ref, gt8_ref, gs8_ref, o_ref):
    pxa = pxa_ref[0]
    pxb = pxb_ref[0]
    xb = x_ref[0]
    adjb = adj_ref[...]
    wf = wf_ref[...]
    tm = tm_ref[...]
    gt = gt_ref[...]
    t8 = t8_ref[...]
    gt8 = gt8_ref[...]
    gs8 = gs8_ref[...]
    f32 = jnp.float32

    xt = jnp.dot(xb, t8, preferred_element_type=f32)
    ea = jnp.exp(xt - pxa)
    eb = jnp.exp(xt - pxb)
    sa = jnp.dot(ea, gs8, preferred_element_type=f32)
    sb = jnp.dot(eb, gs8, preferred_element_type=f32)
    qa = ea * jnp.dot(1.0 / sa, gt8, preferred_element_type=f32)
    qb = eb * jnp.dot(1.0 / sb, gt8, preferred_element_type=f32)

    acc = None
    for g in range(8):
        sl = slice(g * _C, (g + 1) * _C)
        for q_h, px_h in ((qa, pxa), (qb, pxb)):
            term = (jnp.dot(q_h[:, sl], gt, preferred_element_type=f32) *
                    jnp.dot(px_h[:, sl], tm, preferred_element_type=f32))
            acc = term if acc is None else acc + term

    out = jnp.dot(acc, wf, preferred_element_type=f32)
    m = (adjb != 0).astype(f32)
    cnt = jnp.sum(m, axis=1, keepdims=True)
    inv = jnp.where(cnt > 0.0, 1.0 / cnt, 0.0)
    o_ref[0] = jnp.maximum(out * inv, 0.0)


def _tc_dense(px2, x1, adj, wf, tm, gt, t8, gt8, gs8, interpret=False):
    _, v, _ = px2.shape
    vb = 1000
    nj = v // vb
    return pl.pallas_call(
        _tc_body,
        grid=(nj,),
        in_specs=[
            pl.BlockSpec((1, vb, 8 * _C), lambda j: (0, j, 0)),
            pl.BlockSpec((1, vb, 8 * _C), lambda j: (1, j, 0)),
            pl.BlockSpec((1, vb, _C), lambda j: (0, j, 0)),
            pl.BlockSpec((vb, _NB), lambda j: (j, 0)),
            pl.BlockSpec((_NB * _C, _C), lambda j: (0, 0)),
            pl.BlockSpec((_C, _NB * _C), lambda j: (0, 0)),
            pl.BlockSpec((_C, _NB * _C), lambda j: (0, 0)),
            pl.BlockSpec((_C, 8 * _C), lambda j: (0, 0)),
            pl.BlockSpec((8, 8 * _C), lambda j: (0, 0)),
            pl.BlockSpec((8 * _C, 8), lambda j: (0, 0)),
        ],
        out_specs=pl.BlockSpec((1, vb, _C), lambda j: (0, j, 0)),
        out_shape=jax.ShapeDtypeStruct((1, v, _C), jnp.float32),
        interpret=interpret,
    )(px2, px2, x1, adj, wf, tm, gt, t8, gt8, gs8)


def kernel(x, adj, W, u):
    del u
    b, v, c = x.shape
    nb = adj.shape[1]
    out_f = W.shape[2]

    idx = adj.reshape(v, 2, 8).transpose(1, 0, 2).reshape(-1)
    wf = W.transpose(1, 0, 2).reshape(nb * c, out_f)
    eye = jnp.eye(c, dtype=jnp.float32)
    tm = jnp.tile(eye, (1, nb))
    gt = jnp.repeat(eye, nb, axis=1)
    t8 = jnp.tile(eye, (1, 8))
    gt8 = jnp.repeat(jnp.eye(8, dtype=jnp.float32), 16, axis=1)
    gs8 = gt8.T

    pad = jnp.zeros((1, c), x.dtype)
    outs = []
    for bi in range(b):
        table = jnp.concatenate([pad, x[bi]], axis=0)
        px = _sc_gather(table, idx)
        px2 = px.reshape(2, v, 8 * c)
        outs.append(_tc_dense(px2, x[bi:bi + 1], adj, wf, tm, gt, t8, gt8, gs8))
    return jnp.concatenate(outs, axis=0)


# speedup vs baseline: 1.1179x; 1.0686x over previous
import functools

import jax
import jax.numpy as jnp
from jax import lax
from jax.experimental import pallas as pl
from jax.experimental.pallas import tpu as pltpu
from jax.experimental.pallas import tpu_sc as plsc

_NB = 16
_C = 16


def _sc_gather(table, idx_t, v):
    nw = 32
    ch = (2 * v * 8) // nw
    mesh = plsc.VectorSubcoreMesh(core_axis_name="c", subcore_axis_name="s")

    @functools.partial(
        pl.kernel,
        mesh=mesh,
        compiler_params=pltpu.CompilerParams(use_tc_tiling_on_sc=False),
        out_type=jax.ShapeDtypeStruct((2 * v, 8 * _C), jnp.float32),
        scratch_types=[
            pltpu.VMEM((ch,), jnp.int32),
            pltpu.VMEM((ch, _C), jnp.float32),
            pltpu.SemaphoreType.DMA,
        ],
    )
    def k(table_hbm, idx_hbm, out_hbm, idx_v, rows_v, sem):
        wid = lax.axis_index("s") * 2 + lax.axis_index("c")
        nn = wid % _NB
        half = wid // _NB
        v0 = half * ch
        pltpu.sync_copy(idx_hbm.at[pl.ds(nn * v + v0, ch)], idx_v)
        pltpu.async_copy(table_hbm.at[idx_v], rows_v, sem).wait()
        pltpu.sync_copy(
            rows_v,
            out_hbm.at[pl.ds((nn // 8) * v + v0, ch),
                       pl.ds((nn % 8) * _C, _C)],
        )

    return k(table, idx_t)


def _tc_body(pxa_ref, pxb_ref, x_ref, adj_ref, wf_ref, tm_ref, gt_ref,
             t8_ref, gt8_ref, gs8_ref, o_ref):
    pxa = pxa_ref[0]
    pxb = pxb_ref[0]
    xb = x_ref[0]
    adjb = adj_ref[...]
    wf = wf_ref[...]
    tm = tm_ref[...]
    gt = gt_ref[...]
    t8 = t8_ref[...]
    gt8 = gt8_ref[...]
    gs8 = gs8_ref[...]
    f32 = jnp.float32

    xt = jnp.dot(xb, t8, preferred_element_type=f32)
    ea = jnp.exp(xt - pxa)
    eb = jnp.exp(xt - pxb)
    sa = jnp.dot(ea, gs8, preferred_element_type=f32)
    sb = jnp.dot(eb, gs8, preferred_element_type=f32)
    qa = ea * jnp.dot(1.0 / sa, gt8, preferred_element_type=f32)
    qb = eb * jnp.dot(1.0 / sb, gt8, preferred_element_type=f32)

    acc = None
    for g in range(8):
        sl = slice(g * _C, (g + 1) * _C)
        for q_h, px_h in ((qa, pxa), (qb, pxb)):
            term = (jnp.dot(q_h[:, sl], gt, preferred_element_type=f32) *
                    jnp.dot(px_h[:, sl], tm, preferred_element_type=f32))
            acc = term if acc is None else acc + term

    out = jnp.dot(acc, wf, preferred_element_type=f32)
    m = (adjb != 0).astype(f32)
    cnt = jnp.sum(m, axis=1, keepdims=True)
    inv = jnp.where(cnt > 0.0, 1.0 / cnt, 0.0)
    o_ref[0] = jnp.maximum(out * inv, 0.0)


def _tc_dense(px2, x1, adj, wf, tm, gt, t8, gt8, gs8, interpret=False):
    _, v, _ = px2.shape
    vb = 1000
    nj = v // vb
    return pl.pallas_call(
        _tc_body,
        grid=(nj,),
        in_specs=[
            pl.BlockSpec((1, vb, 8 * _C), lambda j: (0, j, 0)),
            pl.BlockSpec((1, vb, 8 * _C), lambda j: (1, j, 0)),
            pl.BlockSpec((1, vb, _C), lambda j: (0, j, 0)),
            pl.BlockSpec((vb, _NB), lambda j: (j, 0)),
            pl.BlockSpec((_NB * _C, _C), lambda j: (0, 0)),
            pl.BlockSpec((_C, _NB * _C), lambda j: (0, 0)),
            pl.BlockSpec((_C, _NB * _C), lambda j: (0, 0)),
            pl.BlockSpec((_C, 8 * _C), lambda j: (0, 0)),
            pl.BlockSpec((8, 8 * _C), lambda j: (0, 0)),
            pl.BlockSpec((8 * _C, 8), lambda j: (0, 0)),
        ],
        out_specs=pl.BlockSpec((1, vb, _C), lambda j: (0, j, 0)),
        out_shape=jax.ShapeDtypeStruct((1, v, _C), jnp.float32),
        interpret=interpret,
    )(px2, px2, x1, adj, wf, tm, gt, t8, gt8, gs8)


def kernel(x, adj, W, u):
    del u
    b, v, c = x.shape
    nb = adj.shape[1]
    out_f = W.shape[2]

    idx = adj.T.reshape(-1)
    wf = W.transpose(1, 0, 2).reshape(nb * c, out_f)
    eye = jnp.eye(c, dtype=jnp.float32)
    tm = jnp.tile(eye, (1, nb))
    gt = jnp.repeat(eye, nb, axis=1)
    t8 = jnp.tile(eye, (1, 8))
    gt8 = jnp.repeat(jnp.eye(8, dtype=jnp.float32), 16, axis=1)
    gs8 = gt8.T

    pad = jnp.zeros((1, c), x.dtype)
    outs = []
    for bi in range(b):
        table = jnp.concatenate([pad, x[bi]], axis=0)
        px = _sc_gather(table, idx, v)
        px2 = px.reshape(2, v, 8 * c)
        outs.append(_tc_dense(px2, x[bi:bi + 1], adj, wf, tm, gt, t8, gt8, gs8))
    return jnp.concatenate(outs, axis=0)


# speedup vs baseline: 1.1527x; 1.0311x over previous
import functools

import jax
import jax.numpy as jnp
from jax import lax
from jax.experimental import pallas as pl
from jax.experimental.pallas import tpu as pltpu
from jax.experimental.pallas import tpu_sc as plsc

_NB = 16
_C = 16


def _sc_gather(table, idx_t, v):
    nw = 32
    ch = (2 * v * 8) // nw
    mesh = plsc.VectorSubcoreMesh(core_axis_name="c", subcore_axis_name="s")

    @functools.partial(
        pl.kernel,
        mesh=mesh,
        compiler_params=pltpu.CompilerParams(use_tc_tiling_on_sc=False),
        out_type=jax.ShapeDtypeStruct((2 * v, 8 * _C), jnp.float32),
        scratch_types=[
            pltpu.VMEM((ch,), jnp.int32),
            pltpu.VMEM((ch, _C), jnp.float32),
            pltpu.SemaphoreType.DMA,
        ],
    )
    def k(table_hbm, idx_hbm, out_hbm, idx_v, rows_v, sem):
        wid = lax.axis_index("s") * 2 + lax.axis_index("c")
        nn = wid % _NB
        half = wid // _NB
        v0 = half * ch
        pltpu.sync_copy(idx_hbm.at[pl.ds(nn * v + v0, ch)], idx_v)
        pltpu.async_copy(table_hbm.at[idx_v], rows_v, sem).wait()
        pltpu.sync_copy(
            rows_v,
            out_hbm.at[pl.ds((nn // 8) * v + v0, ch),
                       pl.ds((nn % 8) * _C, _C)],
        )

    return k(table, idx_t)


def _tc_body(pxa_ref, pxb_ref, x_ref, adj_ref, wf_ref, tm_ref, gt_ref,
             t8_ref, gt8_ref, gs8_ref, o_ref):
    pxa = pxa_ref[0]
    pxb = pxb_ref[0]
    xb = x_ref[0]
    adjb = adj_ref[...]
    wf = wf_ref[...]
    tm = tm_ref[...]
    gt = gt_ref[...]
    t8 = t8_ref[...]
    gt8 = gt8_ref[...]
    gs8 = gs8_ref[...]
    f32 = jnp.float32

    xt = jnp.dot(xb, t8, preferred_element_type=f32)
    ea = jnp.exp(xt - pxa)
    eb = jnp.exp(xt - pxb)
    sa = jnp.dot(ea, gs8, preferred_element_type=f32)
    sb = jnp.dot(eb, gs8, preferred_element_type=f32)
    qa = ea * jnp.dot(1.0 / sa, gt8, preferred_element_type=f32)
    qb = eb * jnp.dot(1.0 / sb, gt8, preferred_element_type=f32)

    acc = None
    for g in range(8):
        sl = slice(g * _C, (g + 1) * _C)
        for q_h, px_h in ((qa, pxa), (qb, pxb)):
            term = (jnp.dot(q_h[:, sl], gt, preferred_element_type=f32) *
                    jnp.dot(px_h[:, sl], tm, preferred_element_type=f32))
            acc = term if acc is None else acc + term

    out = jnp.dot(acc, wf, preferred_element_type=f32)
    m = (adjb != 0).astype(f32)
    cnt = jnp.sum(m, axis=1, keepdims=True)
    inv = jnp.where(cnt > 0.0, 1.0 / cnt, 0.0)
    o_ref[0] = jnp.maximum(out * inv, 0.0)


def _tc_dense(px2, x1, adj, wf, tm, gt, t8, gt8, gs8, interpret=False):
    _, v, _ = px2.shape
    vb = 2000
    nj = v // vb
    return pl.pallas_call(
        _tc_body,
        grid=(nj,),
        in_specs=[
            pl.BlockSpec((1, vb, 8 * _C), lambda j: (0, j, 0)),
            pl.BlockSpec((1, vb, 8 * _C), lambda j: (1, j, 0)),
            pl.BlockSpec((1, vb, _C), lambda j: (0, j, 0)),
            pl.BlockSpec((vb, _NB), lambda j: (j, 0)),
            pl.BlockSpec((_NB * _C, _C), lambda j: (0, 0)),
            pl.BlockSpec((_C, _NB * _C), lambda j: (0, 0)),
            pl.BlockSpec((_C, _NB * _C), lambda j: (0, 0)),
            pl.BlockSpec((_C, 8 * _C), lambda j: (0, 0)),
            pl.BlockSpec((8, 8 * _C), lambda j: (0, 0)),
            pl.BlockSpec((8 * _C, 8), lambda j: (0, 0)),
        ],
        out_specs=pl.BlockSpec((1, vb, _C), lambda j: (0, j, 0)),
        out_shape=jax.ShapeDtypeStruct((1, v, _C), jnp.float32),
        interpret=interpret,
    )(px2, px2, x1, adj, wf, tm, gt, t8, gt8, gs8)


def kernel(x, adj, W, u):
    del u
    b, v, c = x.shape
    nb = adj.shape[1]
    out_f = W.shape[2]

    idx = adj.T.reshape(-1)
    wf = W.transpose(1, 0, 2).reshape(nb * c, out_f)
    eye = jnp.eye(c, dtype=jnp.float32)
    tm = jnp.tile(eye, (1, nb))
    gt = jnp.repeat(eye, nb, axis=1)
    t8 = jnp.tile(eye, (1, 8))
    gt8 = jnp.repeat(jnp.eye(8, dtype=jnp.float32), 16, axis=1)
    gs8 = gt8.T

    pad = jnp.zeros((1, c), x.dtype)
    outs = []
    for bi in range(b):
        table = jnp.concatenate([pad, x[bi]], axis=0)
        px = _sc_gather(table, idx, v)
        px2 = px.reshape(2, v, 8 * c)
        outs.append(_tc_dense(px2, x[bi:bi + 1], adj, wf, tm, gt, t8, gt8, gs8))
    return jnp.concatenate(outs, axis=0)
